# baseline (speedup 1.0000x reference)
import functools

import jax
import jax.numpy as jnp
from jax import lax
from jax.experimental import pallas as pl
from jax.experimental.pallas import tpu as pltpu
from jax.experimental.pallas import tpu_sc as plsc

NC = 2
NS = 16
NW = NC * NS

BATCH = 1024
SEQ = 200
DIM = 64
TOTAL = BATCH * SEQ
PER_W = TOTAL // NW
CHUNK = 128
NCHUNK = PER_W // CHUNK
NBUF = 4


def _body(x_hbm, w_hbm, pe_hbm, out_hbm, idx_v, pe_v, bufs, gsems, ssems):
    wid = lax.axis_index("s") * NC + lax.axis_index("c")

    pltpu.sync_copy(x_hbm.at[wid], idx_v)
    pltpu.sync_copy(pe_hbm, pe_v)

    def fire_gather(j, b):
        pltpu.async_copy(w_hbm.at[idx_v.at[j]], bufs[b], gsems[b])

    for b in range(NBUF - 1):
        fire_gather(b, b)

    def outer(g, carry):
        for b in range(NBUF):
            j = g * NBUF + b
            bn = (b + NBUF - 1) % NBUF

            @pl.when(jnp.logical_and(j + NBUF - 1 <= NCHUNK - 1, j >= 1))
            def _():
                pltpu.make_async_copy(bufs[bn], out_hbm.at[wid, 0],
                                      ssems[bn]).wait()

            @pl.when(j + NBUF - 1 <= NCHUNK - 1)
            def _():
                fire_gather(j + NBUF - 1, bn)

            @pl.when(j <= NCHUNK - 1)
            def _():
                buf = bufs[b]
                pltpu.make_async_copy(w_hbm.at[idx_v.at[j]], buf,
                                      gsems[b]).wait()
                s0 = lax.rem(j * CHUNK, SEQ)

                def addrow(r, c):
                    t = s0 + r
                    t = lax.select(t >= SEQ, t - SEQ, t)
                    for k in range(DIM // 16):
                        sl = pl.ds(k * 16, 16)
                        buf[r, sl] = buf[r, sl] + pe_v[t, sl]
                    return c

                lax.fori_loop(0, CHUNK, addrow, 0)
                pltpu.async_copy(buf, out_hbm.at[wid, j], ssems[b])

        return carry

    n_outer = (NCHUNK + NBUF - 1) // NBUF
    lax.fori_loop(0, n_outer, outer, 0)

    for b in range(NBUF):
        pltpu.make_async_copy(bufs[b], out_hbm.at[wid, 0], ssems[b]).wait()


@jax.jit
def _embed(x3, w, pe2):
    mesh = plsc.VectorSubcoreMesh(core_axis_name="c", subcore_axis_name="s")
    f = pl.kernel(
        _body,
        out_type=jax.ShapeDtypeStruct((NW, NCHUNK, CHUNK, DIM), jnp.float32),
        mesh=mesh,
        scratch_types=dict(
            idx_v=pltpu.VMEM((NCHUNK, CHUNK), jnp.int32),
            pe_v=pltpu.VMEM((SEQ, DIM), jnp.float32),
            bufs=[pltpu.VMEM((CHUNK, DIM), jnp.float32)] * NBUF,
            gsems=[pltpu.SemaphoreType.DMA] * NBUF,
            ssems=[pltpu.SemaphoreType.DMA] * NBUF,
        ),
        compiler_params=pltpu.CompilerParams(use_tc_tiling_on_sc=False),
    )
    return f(x3, w, pe2)


def kernel(x, W, pe):
    x3 = x.astype(jnp.int32).reshape(NW, NCHUNK, CHUNK)
    pe2 = pe.reshape(SEQ, DIM)
    out = _embed(x3, W, pe2)
    return out.reshape(BATCH, SEQ, DIM)


# speedup vs baseline: 1.3978x; 1.3978x over previous
import jax
import jax.numpy as jnp
from jax import lax
from jax.experimental import pallas as pl
from jax.experimental.pallas import tpu as pltpu
from jax.experimental.pallas import tpu_sc as plsc

NC = 2
NS = 16
NW = NC * NS

BATCH = 1024
SEQ = 200
DIM = 64
TOTAL = BATCH * SEQ
ROWS_W = BATCH // NW
LANE = 16
NBUF = 2
NQ = SEQ // LANE
NTAIL = SEQ - NQ * LANE


def _row_fetches(w_hbm, idx, sbuf, sem):
    def fire(q, c):
        iv = idx[pl.ds(q * LANE, LANE)]
        for k in range(LANE):
            pltpu.async_copy(w_hbm.at[iv[k]], sbuf.at[q * LANE + k], sem)
        return c

    lax.fori_loop(0, NQ, fire, 0)
    iv = idx[pl.ds(NQ * LANE, LANE)]
    for k in range(NTAIL):
        pltpu.async_copy(w_hbm.at[iv[k]], sbuf.at[NQ * LANE + k], sem)


def _body(xf_hbm, w_hbm, pef_hbm, out_hbm, pe_v, idxs, sbufs, gsems, ssems):
    wid = lax.axis_index("s") * NC + lax.axis_index("c")
    row0 = wid * ROWS_W

    pltpu.sync_copy(pef_hbm, pe_v)

    def load_idx(item, g):
        pltpu.sync_copy(xf_hbm.at[pl.ds((row0 + item) * SEQ, SEQ)],
                        idxs[g].at[pl.ds(0, SEQ)])

    load_idx(0, 0)
    _row_fetches(w_hbm, idxs[0], sbufs[0], gsems[0])

    def outer(it, carry):
        for b in range(NBUF):
            item = it * NBUF + b
            gn = (b + 1) % NBUF

            @pl.when(item + 1 <= ROWS_W - 1)
            def _():
                load_idx(item + 1, gn)
                @pl.when(item + 1 >= NBUF)
                def _():
                    pltpu.make_async_copy(sbufs[gn], out_hbm.at[0],
                                          ssems[gn]).wait()
                _row_fetches(w_hbm, idxs[gn], sbufs[gn], gsems[gn])

            sbuf = sbufs[b]

            def drain(q, c):
                pltpu.make_async_copy(w_hbm.at[0], sbuf.at[0],
                                      gsems[b]).wait()
                return c

            lax.fori_loop(0, SEQ, drain, 0)

            def addq(q, c):
                for k in range(8):
                    n = q * 8 + k
                    for m in range(DIM // LANE):
                        sl = pl.ds(m * LANE, LANE)
                        sbuf[n, sl] = (sbuf[n, sl]
                                       + pe_v[pl.ds(n * DIM + m * LANE,
                                                    LANE)])
                return c

            lax.fori_loop(0, SEQ // 8, addq, 0)
            pltpu.async_copy(sbuf, out_hbm.at[row0 + item], ssems[b])

        return carry

    lax.fori_loop(0, ROWS_W // NBUF, outer, 0)

    for b in range(NBUF):
        pltpu.make_async_copy(sbufs[b], out_hbm.at[0], ssems[b]).wait()


@jax.jit
def _embed(xf, w, pef):
    mesh = plsc.VectorSubcoreMesh(core_axis_name="c", subcore_axis_name="s")
    f = pl.kernel(
        _body,
        out_type=jax.ShapeDtypeStruct((BATCH, SEQ, DIM), jnp.float32),
        mesh=mesh,
        scratch_types=dict(
            pe_v=pltpu.VMEM((SEQ * DIM,), jnp.float32),
            idxs=[pltpu.VMEM((208,), jnp.int32)] * NBUF,
            sbufs=[pltpu.VMEM((SEQ, DIM), jnp.float32)] * NBUF,
            gsems=[pltpu.SemaphoreType.DMA] * NBUF,
            ssems=[pltpu.SemaphoreType.DMA] * NBUF,
        ),
        compiler_params=pltpu.CompilerParams(
            use_tc_tiling_on_sc=True,
            disable_bounds_checks=True,
        ),
    )
    return f(xf, w, pef)


def kernel(x, W, pe):
    xf = x.astype(jnp.int32).reshape(TOTAL)
    pef = pe.reshape(SEQ * DIM)
    return _embed(xf, W, pef)


# speedup vs baseline: 1.4591x; 1.0438x over previous
import jax
import jax.numpy as jnp
from jax import lax
from jax.experimental import pallas as pl
from jax.experimental.pallas import tpu as pltpu
from jax.experimental.pallas import tpu_sc as plsc

NC = 2
NS = 16
NW = NC * NS

BATCH = 1024
SEQ = 200
DIM = 64
TOTAL = BATCH * SEQ
ROWS_W = BATCH // NW
LANE = 16
NBUF = 2
NQ = SEQ // LANE
NTAIL = SEQ - NQ * LANE


def _row_fetches(w_hbm, idx, sbuf, sem):
    def fire(q, c):
        iv = idx[pl.ds(q * LANE, LANE)]
        for k in range(LANE):
            pltpu.async_copy(w_hbm.at[iv[k]], sbuf.at[q * LANE + k], sem)
        return c

    lax.fori_loop(0, NQ, fire, 0)
    iv = idx[pl.ds(NQ * LANE, LANE)]
    for k in range(NTAIL):
        pltpu.async_copy(w_hbm.at[iv[k]], sbuf.at[NQ * LANE + k], sem)


def _body(xf_hbm, w_hbm, pef_hbm, out_hbm, pe_v, idxs, sbufs, gsems, ssems):
    wid = lax.axis_index("s") * NC + lax.axis_index("c")
    row0 = wid * ROWS_W

    pltpu.sync_copy(pef_hbm, pe_v)

    def load_idx(item, g):
        pltpu.sync_copy(xf_hbm.at[pl.ds((row0 + item) * SEQ, SEQ)],
                        idxs[g].at[pl.ds(0, SEQ)])

    load_idx(0, 0)
    _row_fetches(w_hbm, idxs[0], sbufs[0], gsems[0])

    def outer(it, carry):
        for b in range(NBUF):
            item = it * NBUF + b
            gn = (b + 1) % NBUF

            @pl.when(item + 1 <= ROWS_W - 1)
            def _():
                load_idx(item + 1, gn)
                @pl.when(item + 1 >= NBUF)
                def _():
                    pltpu.make_async_copy(sbufs[gn], out_hbm.at[0],
                                          ssems[gn]).wait()
                _row_fetches(w_hbm, idxs[gn], sbufs[gn], gsems[gn])

            sbuf = sbufs[b]

            pltpu.make_async_copy(w_hbm.at[pl.ds(0, SEQ)], sbuf,
                                  gsems[b]).wait()

            def addq(q, c):
                for k in range(8):
                    n = q * 8 + k
                    for m in range(DIM // LANE):
                        sl = pl.ds(m * LANE, LANE)
                        sbuf[n, sl] = (sbuf[n, sl]
                                       + pe_v[pl.ds(n * DIM + m * LANE,
                                                    LANE)])
                return c

            lax.fori_loop(0, SEQ // 8, addq, 0)
            pltpu.async_copy(sbuf, out_hbm.at[row0 + item], ssems[b])

        return carry

    lax.fori_loop(0, ROWS_W // NBUF, outer, 0)

    for b in range(NBUF):
        pltpu.make_async_copy(sbufs[b], out_hbm.at[0], ssems[b]).wait()


@jax.jit
def _embed(xf, w, pef):
    mesh = plsc.VectorSubcoreMesh(core_axis_name="c", subcore_axis_name="s")
    f = pl.kernel(
        _body,
        out_type=jax.ShapeDtypeStruct((BATCH, SEQ, DIM), jnp.float32),
        mesh=mesh,
        scratch_types=dict(
            pe_v=pltpu.VMEM((SEQ * DIM,), jnp.float32),
            idxs=[pltpu.VMEM((208,), jnp.int32)] * NBUF,
            sbufs=[pltpu.VMEM((SEQ, DIM), jnp.float32)] * NBUF,
            gsems=[pltpu.SemaphoreType.DMA] * NBUF,
            ssems=[pltpu.SemaphoreType.DMA] * NBUF,
        ),
        compiler_params=pltpu.CompilerParams(
            use_tc_tiling_on_sc=True,
            disable_bounds_checks=True,
        ),
    )
    return f(xf, w, pef)


def kernel(x, W, pe):
    xf = x.astype(jnp.int32).reshape(TOTAL)
    pef = pe.reshape(SEQ * DIM)
    return _embed(xf, W, pef)


# speedup vs baseline: 1.4847x; 1.0176x over previous
import jax
import jax.numpy as jnp
from jax import lax
from jax.experimental import pallas as pl
from jax.experimental.pallas import tpu as pltpu
from jax.experimental.pallas import tpu_sc as plsc

NC = 2
NS = 16
NW = NC * NS

BATCH = 1024
SEQ = 200
DIM = 64
TOTAL = BATCH * SEQ
ROWS_W = BATCH // NW
LANE = 16
NBUF = 3
NQ = SEQ // LANE
NTAIL = SEQ - NQ * LANE


def _row_fetches(w_hbm, idx, sbuf, sem):
    def fire(q, c):
        iv = idx[pl.ds(q * LANE, LANE)]
        for k in range(LANE):
            pltpu.async_copy(w_hbm.at[iv[k]], sbuf.at[q * LANE + k], sem)
        return c

    lax.fori_loop(0, NQ, fire, 0)
    iv = idx[pl.ds(NQ * LANE, LANE)]
    for k in range(NTAIL):
        pltpu.async_copy(w_hbm.at[iv[k]], sbuf.at[NQ * LANE + k], sem)


def _body(xf_hbm, w_hbm, pef_hbm, out_hbm, pe_v, idxs, sbufs, gsems, ssems):
    wid = lax.axis_index("s") * NC + lax.axis_index("c")
    row0 = wid * ROWS_W

    pltpu.sync_copy(pef_hbm, pe_v)

    def load_idx(item, g):
        pltpu.sync_copy(xf_hbm.at[pl.ds((row0 + item) * SEQ, SEQ)],
                        idxs[g].at[pl.ds(0, SEQ)])

    load_idx(0, 0)
    _row_fetches(w_hbm, idxs[0], sbufs[0], gsems[0])

    def outer(it, carry):
        for b in range(NBUF):
            item = it * NBUF + b
            gn = (b + 1) % NBUF

            @pl.when(item + 1 <= ROWS_W - 1)
            def _():
                load_idx(item + 1, gn)
                @pl.when(item + 1 >= NBUF)
                def _():
                    pltpu.make_async_copy(sbufs[gn], out_hbm.at[0],
                                          ssems[gn]).wait()
                _row_fetches(w_hbm, idxs[gn], sbufs[gn], gsems[gn])

            @pl.when(item <= ROWS_W - 1)
            def _():
                sbuf = sbufs[b]

                pltpu.make_async_copy(w_hbm.at[pl.ds(0, SEQ)], sbuf,
                                      gsems[b]).wait()

                def addq(q, c):
                    for k in range(8):
                        n = q * 8 + k
                        for m in range(DIM // LANE):
                            sl = pl.ds(m * LANE, LANE)
                            sbuf[n, sl] = (sbuf[n, sl]
                                           + pe_v[pl.ds(n * DIM + m * LANE,
                                                        LANE)])
                    return c

                lax.fori_loop(0, SEQ // 8, addq, 0)
                pltpu.async_copy(sbuf, out_hbm.at[row0 + item], ssems[b])

        return carry

    lax.fori_loop(0, (ROWS_W + NBUF - 1) // NBUF, outer, 0)

    for b in range(NBUF):
        pltpu.make_async_copy(sbufs[b], out_hbm.at[0], ssems[b]).wait()


@jax.jit
def _embed(xf, w, pef):
    mesh = plsc.VectorSubcoreMesh(core_axis_name="c", subcore_axis_name="s")
    f = pl.kernel(
        _body,
        out_type=jax.ShapeDtypeStruct((BATCH, SEQ, DIM), jnp.float32),
        mesh=mesh,
        scratch_types=dict(
            pe_v=pltpu.VMEM((SEQ * DIM,), jnp.float32),
            idxs=[pltpu.VMEM((208,), jnp.int32)] * NBUF,
            sbufs=[pltpu.VMEM((SEQ, DIM), jnp.float32)] * NBUF,
            gsems=[pltpu.SemaphoreType.DMA] * NBUF,
            ssems=[pltpu.SemaphoreType.DMA] * NBUF,
        ),
        compiler_params=pltpu.CompilerParams(
            use_tc_tiling_on_sc=True,
            disable_bounds_checks=True,
        ),
    )
    return f(xf, w, pef)


def kernel(x, W, pe):
    xf = x.astype(jnp.int32).reshape(TOTAL)
    pef = pe.reshape(SEQ * DIM)
    return _embed(xf, W, pef)


# speedup vs baseline: 1.4915x; 1.0046x over previous
import jax
import jax.numpy as jnp
from jax import lax
from jax.experimental import pallas as pl
from jax.experimental.pallas import tpu as pltpu
from jax.experimental.pallas import tpu_sc as plsc

NC = 2
NS = 16
NW = NC * NS

BATCH = 1024
SEQ = 200
DIM = 64
TOTAL = BATCH * SEQ
ROWS_W = BATCH // NW
LANE = 16
NBUF = 3
NQ = SEQ // LANE
NTAIL = SEQ - NQ * LANE


def _row_fetches(w_hbm, idx, sbuf, sem):
    def fire(q, c):
        iv = idx[pl.ds(q * LANE, LANE)]
        for k in range(LANE):
            pltpu.async_copy(w_hbm.at[iv[k]], sbuf.at[q * LANE + k], sem)
        return c

    lax.fori_loop(0, NQ, fire, 0)
    iv = idx[pl.ds(NQ * LANE, LANE)]
    for k in range(NTAIL):
        pltpu.async_copy(w_hbm.at[iv[k]], sbuf.at[NQ * LANE + k], sem)


def _body(xf_hbm, w_hbm, pef_hbm, out_hbm, pe_v, idxs, sbufs, gsems, ssems):
    wid = lax.axis_index("s") * NC + lax.axis_index("c")
    row0 = wid * ROWS_W

    pltpu.sync_copy(pef_hbm, pe_v)

    def load_idx(item, g):
        pltpu.sync_copy(xf_hbm.at[pl.ds((row0 + item) * SEQ, SEQ)],
                        idxs[g].at[pl.ds(0, SEQ)])

    load_idx(0, 0)
    _row_fetches(w_hbm, idxs[0], sbufs[0], gsems[0])

    def outer(it, carry):
        for b in range(NBUF):
            item = it * NBUF + b
            gn = (b + 1) % NBUF
            fire_next = item + 1 <= ROWS_W - 1

            @pl.when(fire_next)
            def _():
                load_idx(item + 1, gn)
                @pl.when(item + 1 >= NBUF)
                def _():
                    pltpu.make_async_copy(sbufs[gn], out_hbm.at[0],
                                          ssems[gn]).wait()

            @pl.when(item <= ROWS_W - 1)
            def _():
                sbuf = sbufs[b]
                sn = sbufs[gn]
                idn = idxs[gn]

                pltpu.make_async_copy(w_hbm.at[pl.ds(0, SEQ)], sbuf,
                                      gsems[b]).wait()

                def addq(q, c):
                    @pl.when(fire_next)
                    def _():
                        iv = idn[pl.ds(q * 8, LANE)]
                        for k in range(8):
                            pltpu.async_copy(w_hbm.at[iv[k]],
                                             sn.at[q * 8 + k], gsems[gn])
                    for k in range(8):
                        n = q * 8 + k
                        for m in range(DIM // LANE):
                            sl = pl.ds(m * LANE, LANE)
                            sbuf[n, sl] = (sbuf[n, sl]
                                           + pe_v[pl.ds(n * DIM + m * LANE,
                                                        LANE)])
                    return c

                lax.fori_loop(0, SEQ // 8, addq, 0)
                pltpu.async_copy(sbuf, out_hbm.at[row0 + item], ssems[b])

        return carry

    lax.fori_loop(0, (ROWS_W + NBUF - 1) // NBUF, outer, 0)

    for b in range(NBUF):
        pltpu.make_async_copy(sbufs[b], out_hbm.at[0], ssems[b]).wait()


@jax.jit
def _embed(xf, w, pef):
    mesh = plsc.VectorSubcoreMesh(core_axis_name="c", subcore_axis_name="s")
    f = pl.kernel(
        _body,
        out_type=jax.ShapeDtypeStruct((BATCH, SEQ, DIM), jnp.float32),
        mesh=mesh,
        scratch_types=dict(
            pe_v=pltpu.VMEM((SEQ * DIM,), jnp.float32),
            idxs=[pltpu.VMEM((208,), jnp.int32)] * NBUF,
            sbufs=[pltpu.VMEM((SEQ, DIM), jnp.float32)] * NBUF,
            gsems=[pltpu.SemaphoreType.DMA] * NBUF,
            ssems=[pltpu.SemaphoreType.DMA] * NBUF,
        ),
        compiler_params=pltpu.CompilerParams(
            use_tc_tiling_on_sc=True,
            disable_bounds_checks=True,
        ),
    )
    return f(xf, w, pef)


def kernel(x, W, pe):
    xf = x.astype(jnp.int32).reshape(TOTAL)
    pef = pe.reshape(SEQ * DIM)
    return _embed(xf, W, pef)
